# baseline (speedup 1.0000x reference)
import functools

import jax
import jax.numpy as jnp
from jax import lax
from jax.experimental import pallas as pl
from jax.experimental.pallas import tpu as pltpu
from jax.experimental.pallas import tpu_sc as plsc

B, L, N, H, V = 4, 512, 2048, 1024, 50265
_PTR_OFF = 10
_EOS_INPUT_ID = 2
_TGT_LO = 50255
_NEG_WORD = float(-1e32)
_NEG_PAD = float(-1e24)

_NC, _NS = 2, 16
_NW = _NC * _NS
_CHUNK = 32


@functools.cache
def _sc_gather(nrows):
    per_w = nrows // _NW

    def body(table_hbm, idx_hbm, out_hbm, idx_v, rows0, rows1,
             gsem0, gsem1, wsem0, wsem1):
        wid = lax.axis_index("s") * _NC + lax.axis_index("c")
        base = wid * per_w
        pltpu.sync_copy(idx_hbm.at[pl.ds(base, per_w)], idx_v)
        bufs, gsems, wsems = (rows0, rows1), (gsem0, gsem1), (wsem0, wsem1)
        nch = per_w // _CHUNK
        hg, hw = [None] * nch, [None] * nch

        def start_gather(c):
            hg[c] = pltpu.async_copy(
                table_hbm.at[idx_v.at[pl.ds(c * _CHUNK, _CHUNK)]],
                bufs[c & 1], gsems[c & 1])

        def start_writeback(c):
            hw[c] = pltpu.async_copy(
                bufs[c & 1], out_hbm.at[pl.ds(base + c * _CHUNK, _CHUNK)],
                wsems[c & 1])

        start_gather(0)
        for c in range(nch):
            if c + 1 < nch:
                if c >= 1:
                    hw[c - 1].wait()
                start_gather(c + 1)
            hg[c].wait()
            start_writeback(c)
        hw[nch - 2].wait()
        hw[nch - 1].wait()

    return pl.kernel(
        body,
        mesh=plsc.VectorSubcoreMesh(core_axis_name="c", subcore_axis_name="s"),
        out_type=jax.ShapeDtypeStruct((nrows, H), jnp.float32),
        scratch_types=[
            pltpu.VMEM((per_w,), jnp.int32),
            pltpu.VMEM((_CHUNK, H), jnp.float32),
            pltpu.VMEM((_CHUNK, H), jnp.float32),
            pltpu.SemaphoreType.DMA,
            pltpu.SemaphoreType.DMA,
            pltpu.SemaphoreType.DMA,
            pltpu.SemaphoreType.DMA,
        ],
    )

_TN = 512
_NBH = 2


def _head_body(lhs_ref, wh_ref, out_ref):
    l = lhs_ref[0]
    h = lax.dot_general(l, wh_ref[...], (((1,), (1,)), ((), ())),
                        preferred_element_type=jnp.float32)
    col = lax.broadcasted_iota(jnp.int32, (L, 16), 1)
    out_ref[0] = jnp.where(col == 0, _NEG_PAD, h)


_head_call = pl.pallas_call(
    _head_body,
    grid=(B,),
    in_specs=[
        pl.BlockSpec((1, L, H), lambda b: (b, 0, 0)),
        pl.BlockSpec((16, H), lambda b: (0, 0)),
    ],
    out_specs=pl.BlockSpec((1, L, 16), lambda b: (b, 0, 0)),
    out_shape=jax.ShapeDtypeStruct((B, L, 16), jnp.float32),
    compiler_params=pltpu.CompilerParams(
        dimension_semantics=("parallel",),
    ),
)


def _tc_body(lhs_ref, g_ref, s_ref, m_ref, *rest):
    out_ref = rest[-1]
    l = lhs_ref[0]
    keys = ((g_ref[0] + s_ref[0]) * 0.5).astype(jnp.bfloat16)
    a = lax.dot_general(l, keys, (((1,), (1,)), ((), ())),
                        preferred_element_type=jnp.float32)
    m = m_ref[0]
    out_ref[0] = jnp.where(m > 0, _NEG_WORD, a)


def _tc_half_call(b_base, aliased):
    in_specs = [
        pl.BlockSpec((1, L, H), lambda b, j: (b + b_base, 0, 0)),
        pl.BlockSpec((1, _TN, H), lambda b, j: (b, j, 0)),
        pl.BlockSpec((1, _TN, H), lambda b, j: (b + b_base, j, 0)),
        pl.BlockSpec((1, 1, _TN), lambda b, j: (b + b_base, 0, j)),
    ]
    kwargs = {}
    if aliased:
        in_specs.append(
            pl.BlockSpec((1, 8, N), lambda b, j: (0, 0, 0)))
        kwargs["input_output_aliases"] = {4: 0}
    return pl.pallas_call(
        _tc_body,
        grid=(_NBH, N // _TN),
        in_specs=in_specs,
        out_specs=pl.BlockSpec((1, L, _TN), lambda b, j: (b + b_base, 0, j)),
        out_shape=jax.ShapeDtypeStruct((B, L, N), jnp.float32),
        compiler_params=pltpu.CompilerParams(
            dimension_semantics=("parallel", "arbitrary"),
        ),
        **kwargs,
    )


def kernel(last_hidden_state, encoder_last_hidden_state, encoder_input_ids,
           encoder_attention_mask, emb_weight):
    ids = encoder_input_ids.astype(jnp.int32)
    nrows_h = _NBH * N
    gather = _sc_gather(nrows_h)
    g0 = gather(emb_weight, ids[:_NBH].reshape(nrows_h)).reshape(_NBH, N, H)
    g1 = gather(emb_weight, ids[_NBH:].reshape(nrows_h)).reshape(_NBH, N, H)

    eos_seen = jnp.cumsum((ids == _EOS_INPUT_ID).astype(jnp.int32), axis=1) >= 1
    maskf = ((encoder_attention_mask == 0) | eos_seen).astype(jnp.float32)
    maskf = maskf.reshape(B, 1, N)

    lhs_bf = last_hidden_state.astype(jnp.bfloat16)
    wh = jnp.concatenate(
        [lax.slice_in_dim(emb_weight, _TGT_LO, _TGT_LO + _PTR_OFF, axis=0),
         jnp.zeros((16 - _PTR_OFF, H), jnp.float32)], axis=0
    ).astype(jnp.bfloat16)

    src = encoder_last_hidden_state
    head = _head_call(lhs_bf, wh)
    word0 = _tc_half_call(0, False)(lhs_bf, g0, src, maskf)
    word = _tc_half_call(_NBH, True)(lhs_bf, g1, src, maskf, word0)
    return jnp.concatenate([head[:, :, :_PTR_OFF], word], axis=-1)


# speedup vs baseline: 1.1053x; 1.1053x over previous
import functools

import jax
import jax.numpy as jnp
from jax import lax
from jax.experimental import pallas as pl
from jax.experimental.pallas import tpu as pltpu
from jax.experimental.pallas import tpu_sc as plsc

B, L, N, H, V = 4, 512, 2048, 1024, 50265
_PTR_OFF = 10
_EOS_INPUT_ID = 2
_TGT_LO = 50255
_NEG_WORD = float(-1e32)
_NEG_PAD = float(-1e24)

_NC, _NS = 2, 16
_NW = _NC * _NS
_CHUNK = 32


@functools.cache
def _sc_gather(nrows):
    per_w = nrows // _NW

    def body(table_hbm, idx_hbm, out_hbm, idx_v, rows0, rows1,
             gsem0, gsem1, wsem0, wsem1):
        wid = lax.axis_index("s") * _NC + lax.axis_index("c")
        base = wid * per_w
        pltpu.sync_copy(idx_hbm.at[pl.ds(base, per_w)], idx_v)
        bufs, gsems, wsems = (rows0, rows1), (gsem0, gsem1), (wsem0, wsem1)
        nch = per_w // _CHUNK
        hg, hw = [None] * nch, [None] * nch

        def start_gather(c):
            hg[c] = pltpu.async_copy(
                table_hbm.at[idx_v.at[pl.ds(c * _CHUNK, _CHUNK)]],
                bufs[c & 1], gsems[c & 1])

        def start_writeback(c):
            hw[c] = pltpu.async_copy(
                bufs[c & 1], out_hbm.at[pl.ds(base + c * _CHUNK, _CHUNK)],
                wsems[c & 1])

        start_gather(0)
        for c in range(nch):
            if c + 1 < nch:
                if c >= 1:
                    hw[c - 1].wait()
                start_gather(c + 1)
            hg[c].wait()
            start_writeback(c)
        hw[nch - 2].wait()
        hw[nch - 1].wait()

    return pl.kernel(
        body,
        mesh=plsc.VectorSubcoreMesh(core_axis_name="c", subcore_axis_name="s"),
        out_type=jax.ShapeDtypeStruct((nrows, H), jnp.float32),
        scratch_types=[
            pltpu.VMEM((per_w,), jnp.int32),
            pltpu.VMEM((_CHUNK, H), jnp.float32),
            pltpu.VMEM((_CHUNK, H), jnp.float32),
            pltpu.SemaphoreType.DMA,
            pltpu.SemaphoreType.DMA,
            pltpu.SemaphoreType.DMA,
            pltpu.SemaphoreType.DMA,
        ],
    )

_TN = 1024
_NBH = 2


def _tc_body(lhs_ref, g_ref, s_ref, m_ref, wh_ref, *rest):
    out_ref = rest[-1]
    j = pl.program_id(1)
    l = lhs_ref[0]
    keys = (g_ref[0] + s_ref[0]).astype(jnp.bfloat16)
    a = lax.dot_general(l, keys, (((1,), (1,)), ((), ())),
                        preferred_element_type=jnp.float32)
    m = m_ref[0]
    masked = jnp.where(m > 0, _NEG_WORD, a)

    @pl.when(j == 0)
    def _head():
        h = lax.dot_general(l, wh_ref[...], (((1,), (1,)), ((), ())),
                            preferred_element_type=jnp.float32)
        col = lax.broadcasted_iota(jnp.int32, (L, 16), 1)
        out_ref[0, :, pl.ds(0, 16)] = jnp.where(col == 0, _NEG_PAD, h)

    for k in range(N // _TN):
        @pl.when(j == k)
        def _store(k=k):
            out_ref[0, :, pl.ds(_PTR_OFF + k * _TN, _TN)] = masked


def _tc_half_call(b_base, aliased):
    in_specs = [
        pl.BlockSpec((1, L, H), lambda b, j: (b + b_base, 0, 0)),
        pl.BlockSpec((1, _TN, H), lambda b, j: (b, j, 0)),
        pl.BlockSpec((1, _TN, H), lambda b, j: (b + b_base, j, 0)),
        pl.BlockSpec((1, 1, _TN), lambda b, j: (b + b_base, 0, j)),
        pl.BlockSpec((16, H), lambda b, j: (0, 0)),
    ]
    kwargs = {}
    if aliased:
        in_specs.append(
            pl.BlockSpec((1, 8, _PTR_OFF + N), lambda b, j: (0, 0, 0)))
        kwargs["input_output_aliases"] = {5: 0}
    return pl.pallas_call(
        _tc_body,
        grid=(_NBH, N // _TN),
        in_specs=in_specs,
        out_specs=pl.BlockSpec((1, L, _PTR_OFF + N),
                               lambda b, j: (b + b_base, 0, 0)),
        out_shape=jax.ShapeDtypeStruct((B, L, _PTR_OFF + N), jnp.float32),
        compiler_params=pltpu.CompilerParams(
            dimension_semantics=("parallel", "arbitrary"),
        ),
        **kwargs,
    )


def kernel(last_hidden_state, encoder_last_hidden_state, encoder_input_ids,
           encoder_attention_mask, emb_weight):
    ids = encoder_input_ids.astype(jnp.int32)
    nrows_h = _NBH * N
    gather = _sc_gather(nrows_h)
    g0 = gather(emb_weight, ids[:_NBH].reshape(nrows_h)).reshape(_NBH, N, H)
    g1 = gather(emb_weight, ids[_NBH:].reshape(nrows_h)).reshape(_NBH, N, H)

    eos_seen = jnp.cumsum((ids == _EOS_INPUT_ID).astype(jnp.int32), axis=1) >= 1
    maskf = ((encoder_attention_mask == 0) | eos_seen).astype(jnp.float32)
    maskf = maskf.reshape(B, 1, N)

    lhs_bf = (last_hidden_state * 0.5).astype(jnp.bfloat16)
    wh = jnp.concatenate(
        [lax.slice_in_dim(emb_weight, _TGT_LO, _TGT_LO + _PTR_OFF, axis=0) * 2.0,
         jnp.zeros((16 - _PTR_OFF, H), jnp.float32)], axis=0
    ).astype(jnp.bfloat16)

    src = encoder_last_hidden_state
    logits0 = _tc_half_call(0, False)(lhs_bf, g0, src, maskf, wh)
    return _tc_half_call(_NBH, True)(lhs_bf, g1, src, maskf, wh, logits0)
